# Initial kernel scaffold; baseline (speedup 1.0000x reference)
import jax
import jax.numpy as jnp
from jax.experimental import pallas as pl


def kernel(z, edge_index, z_table, b1, b2, b3, w_c1, b_c1, w_c2, b_c2, w_l1, b_l1, w_l2, b_l2):
    raise NotImplementedError("write your pallas kernel here")


# speedup vs baseline: 3.4039x; 3.4039x over previous
import numpy as np

import jax
import jax.numpy as jnp
from jax import lax
from jax.experimental import pallas as pl
from jax.experimental.pallas import tpu as pltpu
from jax.experimental.pallas import tpu_sc as plsc

N = 100000
E = 1600000
H = 32
HH = 16
K = 30
HALF = 50000
SC_ROWS = 51200
NR = 2 * SC_ROWS
TRASH = 50048
ROWS_PT = 3200
CHUNK = 128
CPB = 24
EPAD = 1622016
ECHUNKS = EPAD // CHUNK
NBLK = EPAD // (CPB * CHUNK)
BPT = NBLK // 16
NB = 4

_mesh = plsc.VectorSubcoreMesh(core_axis_name="c", subcore_axis_name="s")


def _map_rows(src_ref, dst_ref, fn):
    def body(r, _):
        for q in range(CHUNK // 16):
            dst_ref[r, pl.ds(q * 16, 16)] = fn(src_ref[r, pl.ds(q * 16, 16)])
        return 0
    lax.fori_loop(0, CPB, body, 0)


def _rho(v):
    r = jnp.where(v >= HALF, v + 1200, v)
    return jnp.where(v >= N, TRASH, r)


def _prep_body(src_hbm, dst_hbm, deg_out_hbm, deg_in_hbm, srcr_hbm, dlc_hbm,
               iblk, ablk, bblk, cblk, ones_v, zero_v, deg_sp,
               ssem, osem, dsem):
    c = lax.axis_index("c")
    s = lax.axis_index("s")

    def zbody(i, _):
        zero_v[pl.ds(i * 16, 16)] = jnp.zeros((16,), jnp.float32)
        return 0
    lax.fori_loop(0, zero_v.shape[0] // 16, zbody, 0, unroll=8)

    def obody(i, _):
        ones_v[pl.ds(i * 16, 16)] = jnp.ones((16,), jnp.float32)
        return 0
    lax.fori_loop(0, CHUNK // 16, obody, 0)

    pltpu.sync_copy(zero_v, deg_sp.at[pl.ds(s * (NR // 16), NR // 16)])
    plsc.subcore_barrier()

    def blk(b, _):
        brow = (s * BPT + b) * CPB

        @pl.when(c == 0)
        def _():
            pltpu.sync_copy(src_hbm.at[pl.ds(brow, CPB)], iblk)
            _map_rows(iblk, ablk, _rho)
            cp = pltpu.async_copy(ablk, srcr_hbm.at[pl.ds(brow, CPB)], osem)
            for j in range(CPB):
                pltpu.async_copy(ones_v, deg_sp.at[ablk.at[j]], dsem,
                                 add=True)
            for j in range(CPB):
                pltpu.make_async_copy(ones_v, deg_sp.at[ablk.at[j]],
                                      dsem).wait()
            cp.wait()

        @pl.when(c != 0)
        def _():
            pltpu.sync_copy(dst_hbm.at[pl.ds(brow, CPB)], iblk)
            _map_rows(iblk, ablk, lambda v: jnp.where(v < HALF, v, TRASH))
            cp0 = pltpu.async_copy(ablk, dlc_hbm.at[0, pl.ds(brow, CPB)],
                                   osem)
            _map_rows(iblk, bblk,
                      lambda v: jnp.where((v >= HALF) & (v < N), v - HALF,
                                          TRASH))
            cp1 = pltpu.async_copy(bblk, dlc_hbm.at[1, pl.ds(brow, CPB)],
                                   ssem)
            _map_rows(iblk, cblk, _rho)
            for j in range(CPB):
                pltpu.async_copy(ones_v, deg_sp.at[cblk.at[j]], dsem,
                                 add=True)
            for j in range(CPB):
                pltpu.make_async_copy(ones_v, deg_sp.at[cblk.at[j]],
                                      dsem).wait()
            cp0.wait()
            cp1.wait()
        return 0

    lax.fori_loop(0, BPT, blk, 0)
    plsc.subcore_barrier()

    @pl.when(c == 0)
    def _():
        pltpu.sync_copy(deg_sp.at[pl.ds(s * (NR // 16), NR // 16)],
                        deg_out_hbm.at[pl.ds(s * (NR // 16), NR // 16)])

    @pl.when(c != 0)
    def _():
        pltpu.sync_copy(deg_sp.at[pl.ds(s * (NR // 16), NR // 16)],
                        deg_in_hbm.at[pl.ds(s * (NR // 16), NR // 16)])


_prep = pl.kernel(
    _prep_body,
    out_type=(
        jax.ShapeDtypeStruct((NR,), jnp.float32),
        jax.ShapeDtypeStruct((NR,), jnp.float32),
        jax.ShapeDtypeStruct((ECHUNKS, CHUNK), jnp.int32),
        jax.ShapeDtypeStruct((2, ECHUNKS, CHUNK), jnp.int32),
    ),
    mesh=_mesh,
    compiler_params=pltpu.CompilerParams(use_tc_tiling_on_sc=False),
    scratch_types=[
        pltpu.VMEM((CPB, CHUNK), jnp.int32),
        pltpu.VMEM((CPB, CHUNK), jnp.int32),
        pltpu.VMEM((CPB, CHUNK), jnp.int32),
        pltpu.VMEM((CPB, CHUNK), jnp.int32),
        pltpu.VMEM((CHUNK,), jnp.float32),
        pltpu.VMEM((NR // 16,), jnp.float32),
        pltpu.VMEM_SHARED((NR,), jnp.float32),
        pltpu.SemaphoreType.DMA,
        pltpu.SemaphoreType.DMA,
        pltpu.SemaphoreType.DMA,
    ],
)


def _embed_body(z_hbm, tab_hbm, h0_hbm, zb, rows, gsem):
    c = lax.axis_index("c")
    s = lax.axis_index("s")
    nbase = c * HALF + s * ROWS_PT
    rbase = c * SC_ROWS + s * ROWS_PT

    def body(j, _):
        pltpu.sync_copy(z_hbm.at[pl.ds(nbase + j * CHUNK, CHUNK)], zb)
        pltpu.async_copy(tab_hbm.at[zb], rows, gsem).wait()
        pltpu.sync_copy(rows, h0_hbm.at[pl.ds(rbase + j * CHUNK, CHUNK)])
        return 0

    lax.fori_loop(0, ROWS_PT // CHUNK, body, 0)


_embed = pl.kernel(
    _embed_body,
    out_type=jax.ShapeDtypeStruct((NR, H), jnp.float32),
    mesh=_mesh,
    compiler_params=pltpu.CompilerParams(use_tc_tiling_on_sc=False),
    scratch_types=[
        pltpu.VMEM((CHUNK,), jnp.int32),
        pltpu.VMEM((CHUNK, H), jnp.float32),
        pltpu.SemaphoreType.DMA,
    ],
)


def _agg_body(hn_hbm, srcr_hbm, dlc_hbm, out_hbm,
              sblk, dblk, rows, zb, acc, isem, gsem, ssem):
    c = lax.axis_index("c")
    s = lax.axis_index("s")

    def zbody(i, _):
        zb[i, pl.ds(0, 16)] = jnp.zeros((16,), jnp.float32)
        return 0
    lax.fori_loop(0, 400, zbody, 0, unroll=8)
    for k in range(8):
        pltpu.sync_copy(zb, acc.at[pl.ds(s * ROWS_PT + k * 400, 400)])
    plsc.subcore_barrier()

    cbase = s * BPT * CPB

    def gather(j):
        pltpu.async_copy(hn_hbm.at[sblk.at[j]], rows.at[j % NB], gsem)

    def gwait(j):
        pltpu.make_async_copy(hn_hbm.at[sblk.at[j]], rows.at[j % NB],
                              gsem).wait()

    def scat(j):
        pltpu.async_copy(rows.at[j % NB], acc.at[dblk.at[j]], ssem, add=True)

    def swait(j):
        pltpu.make_async_copy(rows.at[j % NB], acc.at[dblk.at[j]],
                              ssem).wait()

    def blk(b, _):
        crow = cbase + b * CPB
        c1 = pltpu.async_copy(srcr_hbm.at[pl.ds(crow, CPB)], sblk, isem)
        c2 = pltpu.async_copy(dlc_hbm.at[c, pl.ds(crow, CPB)], dblk, isem)
        c1.wait()
        c2.wait()
        for j in range(CPB):
            if j >= NB:
                swait(j - NB)
            gather(j)
            if j >= 1:
                gwait(j - 1)
                scat(j - 1)
        gwait(CPB - 1)
        scat(CPB - 1)
        for j in range(CPB - NB, CPB):
            swait(j)
        return 0

    lax.fori_loop(0, BPT, blk, 0)

    plsc.subcore_barrier()
    pltpu.sync_copy(acc.at[pl.ds(s * ROWS_PT, ROWS_PT)],
                    out_hbm.at[pl.ds(c * SC_ROWS + s * ROWS_PT, ROWS_PT)])


_agg = pl.kernel(
    _agg_body,
    out_type=jax.ShapeDtypeStruct((NR, HH), jnp.float32),
    mesh=_mesh,
    compiler_params=pltpu.CompilerParams(use_tc_tiling_on_sc=False),
    scratch_types=[
        pltpu.VMEM((CPB, CHUNK), jnp.int32),
        pltpu.VMEM((CPB, CHUNK), jnp.int32),
        pltpu.VMEM((NB, CHUNK, HH), jnp.float32),
        pltpu.VMEM((400, HH), jnp.float32),
        pltpu.VMEM_SHARED((SC_ROWS, HH), jnp.float32),
        pltpu.SemaphoreType.DMA,
        pltpu.SemaphoreType.DMA,
        pltpu.SemaphoreType.DMA,
    ],
)


_RB = 1024


def _norm0_body(h0_ref, dego_ref, lo_ref, hi_ref):
    cout = lax.rsqrt(jnp.maximum(dego_ref[...], 1.0))
    hn = h0_ref[...] * cout
    lo_ref[...] = hn[:, 0:HH]
    hi_ref[...] = hn[:, HH:H]


def _norm_body(alo_ref, ahi_ref, degi_ref, dego_ref, b_ref,
               xlo_ref, xhi_ref, hlo_ref, hhi_ref):
    cin = lax.rsqrt(jnp.maximum(degi_ref[...], 1.0))
    cout = lax.rsqrt(jnp.maximum(dego_ref[...], 1.0))
    xlo = jnp.tanh(alo_ref[...] * cin + b_ref[:, 0:HH])
    xhi = jnp.tanh(ahi_ref[...] * cin + b_ref[:, HH:H])
    xlo_ref[...] = xlo
    xhi_ref[...] = xhi
    hlo_ref[...] = xlo * cout
    hhi_ref[...] = xhi * cout


def _norm3_body(alo_ref, ahi_ref, degi_ref, b_ref, xlo_ref, xhi_ref, key_ref):
    cin = lax.rsqrt(jnp.maximum(degi_ref[...], 1.0))
    xlo_ref[...] = jnp.tanh(alo_ref[...] * cin + b_ref[:, 0:HH])
    xhi = jnp.tanh(ahi_ref[...] * cin + b_ref[:, HH:H])
    xhi_ref[...] = xhi
    key_ref[...] = xhi[:, HH - 1:HH]


_row_spec = pl.BlockSpec((_RB, H), lambda i: (i, 0))
_half_spec = pl.BlockSpec((_RB, HH), lambda i: (i, 0))
_deg_spec = pl.BlockSpec((_RB, 1), lambda i: (i, 0))
_b_spec = pl.BlockSpec((1, H), lambda i: (0, 0))
_f32 = jnp.float32
_half_sds = jax.ShapeDtypeStruct((NR, HH), _f32)


def _norm0(h0, deg_out):
    return pl.pallas_call(
        _norm0_body,
        grid=(NR // _RB,),
        in_specs=[_row_spec, _deg_spec],
        out_specs=[_half_spec, _half_spec],
        out_shape=[_half_sds, _half_sds],
    )(h0, deg_out.reshape(NR, 1))


def _norm(alo, ahi, deg_in, deg_out, b):
    return pl.pallas_call(
        _norm_body,
        grid=(NR // _RB,),
        in_specs=[_half_spec, _half_spec, _deg_spec, _deg_spec, _b_spec],
        out_specs=[_half_spec] * 4,
        out_shape=[_half_sds] * 4,
    )(alo, ahi, deg_in.reshape(NR, 1), deg_out.reshape(NR, 1), b.reshape(1, H))


def _norm3(alo, ahi, deg_in, b):
    return pl.pallas_call(
        _norm3_body,
        grid=(NR // _RB,),
        in_specs=[_half_spec, _half_spec, _deg_spec, _b_spec],
        out_specs=[_half_spec, _half_spec, _deg_spec],
        out_shape=[_half_sds, _half_sds,
                   jax.ShapeDtypeStruct((NR, 1), _f32)],
    )(alo, ahi, deg_in.reshape(NR, 1), b.reshape(1, H))


def _topk_body(keys_ref, idx_ref):
    kr, kc = NR // 128, 128
    flat_iota = (lax.broadcasted_iota(jnp.int32, (kr, kc), 0) * kc
                 + lax.broadcasted_iota(jnp.int32, (kr, kc), 1))
    valid = ((flat_iota < HALF)
             | ((flat_iota >= SC_ROWS) & (flat_iota < SC_ROWS + HALF)))
    keys = jnp.where(valid, keys_ref[...], -jnp.inf)
    for k in range(K):
        m = jnp.max(keys)
        idx = jnp.min(jnp.where(keys == m, flat_iota, NR))
        idx_ref[0, k] = idx
        keys = jnp.where(flat_iota == idx, -jnp.inf, keys)
    for k in range(K, CHUNK):
        idx_ref[0, k] = 0


def _topk(keys):
    return pl.pallas_call(
        _topk_body,
        in_specs=[pl.BlockSpec(memory_space=pltpu.VMEM)],
        out_specs=pl.BlockSpec(memory_space=pltpu.SMEM),
        out_shape=jax.ShapeDtypeStruct((1, CHUNK), jnp.int32),
    )(keys)


def _gather_body(idx_hbm, x1l, x1h, x2l, x2h, x3l, x3h, out_hbm,
                 idxv, rows, gsem):
    c = lax.axis_index("c")
    s = lax.axis_index("s")

    @pl.when((c == 0) & (s == 0))
    def _():
        pltpu.sync_copy(idx_hbm.at[0], idxv)
        for q, src in enumerate((x1l, x1h, x2l, x2h, x3l, x3h)):
            pltpu.async_copy(src.at[idxv], rows, gsem).wait()
            pltpu.sync_copy(rows, out_hbm.at[q])


_gather_topk = pl.kernel(
    _gather_body,
    out_type=jax.ShapeDtypeStruct((6, CHUNK, HH), jnp.float32),
    mesh=_mesh,
    compiler_params=pltpu.CompilerParams(use_tc_tiling_on_sc=False),
    scratch_types=[
        pltpu.VMEM((CHUNK,), jnp.int32),
        pltpu.VMEM((CHUNK, HH), jnp.float32),
        pltpu.SemaphoreType.DMA,
    ],
)


def _head_body(tk_ref, w1t_ref, bc1_ref, w2r_ref, bc2_ref, wl1_ref,
               bl1_ref, wl2_ref, bl2_ref, out_ref):
    t = jnp.concatenate([tk_ref[q, 0:K, :] for q in range(6)], axis=1)
    c1 = jnp.maximum(
        jnp.dot(t, w1t_ref[...], preferred_element_type=_f32)
        + bc1_ref[...], 0.0)
    mp = jnp.max(c1.reshape(K // 2, 2, 16), axis=1)
    w = jnp.concatenate([mp[kw:kw + 11, :] for kw in range(5)], axis=1)
    c2 = jnp.maximum(
        jnp.dot(w, w2r_ref[...], preferred_element_type=_f32)
        + bc2_ref[...], 0.0)
    h1 = bl1_ref[...]
    for tpos in range(11):
        h1 = h1 + jnp.dot(c2[tpos:tpos + 1, :], wl1_ref[tpos],
                          preferred_element_type=_f32)
    h1 = jnp.maximum(h1, 0.0)
    out_ref[...] = (jnp.dot(h1, wl2_ref[...], preferred_element_type=_f32)
                    + bl2_ref[...])


def _head(tk, w1t, bc1, w2r, bc2, wl1p3, bl1, wl2, bl2):
    return pl.pallas_call(
        _head_body,
        out_shape=jax.ShapeDtypeStruct((1, 1), _f32),
    )(tk, w1t, bc1, w2r, bc2, wl1p3, bl1, wl2, bl2)


def kernel(z, edge_index, z_table, b1, b2, b3, w_c1, b_c1, w_c2, b_c2,
           w_l1, b_l1, w_l2, b_l2):
    src = jnp.pad(edge_index[0].astype(jnp.int32), (0, EPAD - E),
                  constant_values=N).reshape(ECHUNKS, CHUNK)
    dst = jnp.pad(edge_index[1].astype(jnp.int32), (0, EPAD - E),
                  constant_values=N).reshape(ECHUNKS, CHUNK)
    zp = jnp.pad(z.astype(jnp.int32), (0, NR - N))

    deg_out, deg_in, srcr, dlc = _prep(src, dst)

    h0 = _embed(zp, z_table)
    hlo, hhi = _norm0(h0, deg_out)

    a1l = _agg(hlo, srcr, dlc)
    a1h = _agg(hhi, srcr, dlc)
    x1l, x1h, hlo, hhi = _norm(a1l, a1h, deg_in, deg_out, b1)
    a2l = _agg(hlo, srcr, dlc)
    a2h = _agg(hhi, srcr, dlc)
    x2l, x2h, hlo, hhi = _norm(a2l, a2h, deg_in, deg_out, b2)
    a3l = _agg(hlo, srcr, dlc)
    a3h = _agg(hhi, srcr, dlc)
    x3l, x3h, keys = _norm3(a3l, a3h, deg_in, b3)

    w1t = w_c1[:, 0, :].T
    w2r = jnp.transpose(w_c2, (2, 1, 0)).reshape(80, 32)
    pos = np.arange(352)
    wl1p3 = w_l1[(pos % 32) * 11 + pos // 32, :].reshape(11, 32, 128)

    idx = _topk(keys.reshape(NR // 128, 128))
    tk = _gather_topk(idx, x1l, x1h, x2l, x2h, x3l, x3h)
    return _head(tk, w1t, b_c1.reshape(1, 16), w2r, b_c2.reshape(1, 32),
                 wl1p3, b_l1.reshape(1, 128), w_l2, b_l2.reshape(1, 1))


# speedup vs baseline: 3.4183x; 1.0042x over previous
import numpy as np

import jax
import jax.numpy as jnp
from jax import lax
from jax.experimental import pallas as pl
from jax.experimental.pallas import tpu as pltpu
from jax.experimental.pallas import tpu_sc as plsc

N = 100000
E = 1600000
H = 32
HH = 16
K = 30
HALF = 50000
SC_ROWS = 51200
NR = 2 * SC_ROWS
TRASH = 50048
ROWS_PT = 3200
CHUNK = 128
CPB = 88
EPAD = 1622016
ECHUNKS = EPAD // CHUNK
NBLK = EPAD // (CPB * CHUNK)
BPT = NBLK // 16
NB = 8

_mesh = plsc.VectorSubcoreMesh(core_axis_name="c", subcore_axis_name="s")


def _map_rows(src_ref, dst_ref, fn):
    def body(r, _):
        for q in range(CHUNK // 16):
            dst_ref[r, pl.ds(q * 16, 16)] = fn(src_ref[r, pl.ds(q * 16, 16)])
        return 0
    lax.fori_loop(0, CPB, body, 0)


def _rho(v):
    r = jnp.where(v >= HALF, v + 1200, v)
    return jnp.where(v >= N, TRASH, r)


def _prep_body(src_hbm, dst_hbm, deg_out_hbm, deg_in_hbm, srcr_hbm, dlc_hbm,
               iblk, ablk, bblk, cblk, ones_v, zero_v, deg_sp,
               ssem, osem, dsem):
    c = lax.axis_index("c")
    s = lax.axis_index("s")

    def zbody(i, _):
        zero_v[pl.ds(i * 16, 16)] = jnp.zeros((16,), jnp.float32)
        return 0
    lax.fori_loop(0, zero_v.shape[0] // 16, zbody, 0, unroll=8)

    def obody(i, _):
        ones_v[pl.ds(i * 16, 16)] = jnp.ones((16,), jnp.float32)
        return 0
    lax.fori_loop(0, CHUNK // 16, obody, 0)

    pltpu.sync_copy(zero_v, deg_sp.at[pl.ds(s * (NR // 16), NR // 16)])
    plsc.subcore_barrier()

    def blk(b, _):
        brow = (s * BPT + b) * CPB

        @pl.when(c == 0)
        def _():
            pltpu.sync_copy(src_hbm.at[pl.ds(brow, CPB)], iblk)
            _map_rows(iblk, ablk, _rho)
            cp = pltpu.async_copy(ablk, srcr_hbm.at[pl.ds(brow, CPB)], osem)
            for j in range(CPB):
                pltpu.async_copy(ones_v, deg_sp.at[ablk.at[j]], dsem,
                                 add=True)
            for j in range(CPB):
                pltpu.make_async_copy(ones_v, deg_sp.at[ablk.at[j]],
                                      dsem).wait()
            cp.wait()

        @pl.when(c != 0)
        def _():
            pltpu.sync_copy(dst_hbm.at[pl.ds(brow, CPB)], iblk)
            _map_rows(iblk, ablk, lambda v: jnp.where(v < HALF, v, TRASH))
            cp0 = pltpu.async_copy(ablk, dlc_hbm.at[0, pl.ds(brow, CPB)],
                                   osem)
            _map_rows(iblk, bblk,
                      lambda v: jnp.where((v >= HALF) & (v < N), v - HALF,
                                          TRASH))
            cp1 = pltpu.async_copy(bblk, dlc_hbm.at[1, pl.ds(brow, CPB)],
                                   ssem)
            _map_rows(iblk, cblk, _rho)
            for j in range(CPB):
                pltpu.async_copy(ones_v, deg_sp.at[cblk.at[j]], dsem,
                                 add=True)
            for j in range(CPB):
                pltpu.make_async_copy(ones_v, deg_sp.at[cblk.at[j]],
                                      dsem).wait()
            cp0.wait()
            cp1.wait()
        return 0

    lax.fori_loop(0, BPT, blk, 0)
    plsc.subcore_barrier()

    @pl.when(c == 0)
    def _():
        pltpu.sync_copy(deg_sp.at[pl.ds(s * (NR // 16), NR // 16)],
                        deg_out_hbm.at[pl.ds(s * (NR // 16), NR // 16)])

    @pl.when(c != 0)
    def _():
        pltpu.sync_copy(deg_sp.at[pl.ds(s * (NR // 16), NR // 16)],
                        deg_in_hbm.at[pl.ds(s * (NR // 16), NR // 16)])


_prep = pl.kernel(
    _prep_body,
    out_type=(
        jax.ShapeDtypeStruct((NR,), jnp.float32),
        jax.ShapeDtypeStruct((NR,), jnp.float32),
        jax.ShapeDtypeStruct((ECHUNKS, CHUNK), jnp.int32),
        jax.ShapeDtypeStruct((2, ECHUNKS, CHUNK), jnp.int32),
    ),
    mesh=_mesh,
    compiler_params=pltpu.CompilerParams(use_tc_tiling_on_sc=False),
    scratch_types=[
        pltpu.VMEM((CPB, CHUNK), jnp.int32),
        pltpu.VMEM((CPB, CHUNK), jnp.int32),
        pltpu.VMEM((CPB, CHUNK), jnp.int32),
        pltpu.VMEM((CPB, CHUNK), jnp.int32),
        pltpu.VMEM((CHUNK,), jnp.float32),
        pltpu.VMEM((NR // 16,), jnp.float32),
        pltpu.VMEM_SHARED((NR,), jnp.float32),
        pltpu.SemaphoreType.DMA,
        pltpu.SemaphoreType.DMA,
        pltpu.SemaphoreType.DMA,
    ],
)


def _embed_body(z_hbm, tab_hbm, h0_hbm, zb, rows, gsem):
    c = lax.axis_index("c")
    s = lax.axis_index("s")
    nbase = c * HALF + s * ROWS_PT
    rbase = c * SC_ROWS + s * ROWS_PT

    def body(j, _):
        pltpu.sync_copy(z_hbm.at[pl.ds(nbase + j * CHUNK, CHUNK)], zb)
        pltpu.async_copy(tab_hbm.at[zb], rows, gsem).wait()
        pltpu.sync_copy(rows, h0_hbm.at[pl.ds(rbase + j * CHUNK, CHUNK)])
        return 0

    lax.fori_loop(0, ROWS_PT // CHUNK, body, 0)


_embed = pl.kernel(
    _embed_body,
    out_type=jax.ShapeDtypeStruct((NR, H), jnp.float32),
    mesh=_mesh,
    compiler_params=pltpu.CompilerParams(use_tc_tiling_on_sc=False),
    scratch_types=[
        pltpu.VMEM((CHUNK,), jnp.int32),
        pltpu.VMEM((CHUNK, H), jnp.float32),
        pltpu.SemaphoreType.DMA,
    ],
)


def _agg_body(hn_hbm, srcr_hbm, dlc_hbm, out_hbm,
              sblk, dblk, rows, zb, acc, isem, gsem, ssem):
    c = lax.axis_index("c")
    s = lax.axis_index("s")

    def zbody(i, _):
        zb[i, pl.ds(0, 16)] = jnp.zeros((16,), jnp.float32)
        return 0
    lax.fori_loop(0, 400, zbody, 0, unroll=8)
    for k in range(8):
        pltpu.sync_copy(zb, acc.at[pl.ds(s * ROWS_PT + k * 400, 400)])
    plsc.subcore_barrier()

    cbase = s * BPT * CPB

    def gather(j):
        pltpu.async_copy(hn_hbm.at[sblk.at[j]], rows.at[j % NB], gsem)

    def gwait(j):
        pltpu.make_async_copy(hn_hbm.at[sblk.at[j]], rows.at[j % NB],
                              gsem).wait()

    def scat(j):
        pltpu.async_copy(rows.at[j % NB], acc.at[dblk.at[j]], ssem, add=True)

    def swait(j):
        pltpu.make_async_copy(rows.at[j % NB], acc.at[dblk.at[j]],
                              ssem).wait()

    def blk(b, _):
        crow = cbase + b * CPB
        c1 = pltpu.async_copy(srcr_hbm.at[pl.ds(crow, CPB)], sblk, isem)
        c2 = pltpu.async_copy(dlc_hbm.at[c, pl.ds(crow, CPB)], dblk, isem)
        c1.wait()
        c2.wait()
        for j in range(NB):
            gather(j)
        for j in range(CPB):
            gwait(j)
            scat(j)
            if j + NB < CPB:
                swait(j)
                gather(j + NB)
        for j in range(CPB - NB, CPB):
            swait(j)
        return 0

    lax.fori_loop(0, BPT, blk, 0)

    plsc.subcore_barrier()
    pltpu.sync_copy(acc.at[pl.ds(s * ROWS_PT, ROWS_PT)],
                    out_hbm.at[pl.ds(c * SC_ROWS + s * ROWS_PT, ROWS_PT)])


_agg = pl.kernel(
    _agg_body,
    out_type=jax.ShapeDtypeStruct((NR, HH), jnp.float32),
    mesh=_mesh,
    compiler_params=pltpu.CompilerParams(use_tc_tiling_on_sc=False),
    scratch_types=[
        pltpu.VMEM((CPB, CHUNK), jnp.int32),
        pltpu.VMEM((CPB, CHUNK), jnp.int32),
        pltpu.VMEM((NB, CHUNK, HH), jnp.float32),
        pltpu.VMEM((400, HH), jnp.float32),
        pltpu.VMEM_SHARED((SC_ROWS, HH), jnp.float32),
        pltpu.SemaphoreType.DMA,
        pltpu.SemaphoreType.DMA,
        pltpu.SemaphoreType.DMA,
    ],
)


_RB = 1024


def _norm0_body(h0_ref, dego_ref, lo_ref, hi_ref):
    cout = lax.rsqrt(jnp.maximum(dego_ref[...], 1.0))
    hn = h0_ref[...] * cout
    lo_ref[...] = hn[:, 0:HH]
    hi_ref[...] = hn[:, HH:H]


def _norm_body(alo_ref, ahi_ref, degi_ref, dego_ref, b_ref,
               xlo_ref, xhi_ref, hlo_ref, hhi_ref):
    cin = lax.rsqrt(jnp.maximum(degi_ref[...], 1.0))
    cout = lax.rsqrt(jnp.maximum(dego_ref[...], 1.0))
    xlo = jnp.tanh(alo_ref[...] * cin + b_ref[:, 0:HH])
    xhi = jnp.tanh(ahi_ref[...] * cin + b_ref[:, HH:H])
    xlo_ref[...] = xlo
    xhi_ref[...] = xhi
    hlo_ref[...] = xlo * cout
    hhi_ref[...] = xhi * cout


def _norm3_body(alo_ref, ahi_ref, degi_ref, b_ref, xlo_ref, xhi_ref, key_ref):
    cin = lax.rsqrt(jnp.maximum(degi_ref[...], 1.0))
    xlo_ref[...] = jnp.tanh(alo_ref[...] * cin + b_ref[:, 0:HH])
    xhi = jnp.tanh(ahi_ref[...] * cin + b_ref[:, HH:H])
    xhi_ref[...] = xhi
    key_ref[...] = xhi[:, HH - 1:HH]


_row_spec = pl.BlockSpec((_RB, H), lambda i: (i, 0))
_half_spec = pl.BlockSpec((_RB, HH), lambda i: (i, 0))
_deg_spec = pl.BlockSpec((_RB, 1), lambda i: (i, 0))
_b_spec = pl.BlockSpec((1, H), lambda i: (0, 0))
_f32 = jnp.float32
_half_sds = jax.ShapeDtypeStruct((NR, HH), _f32)


def _norm0(h0, deg_out):
    return pl.pallas_call(
        _norm0_body,
        grid=(NR // _RB,),
        in_specs=[_row_spec, _deg_spec],
        out_specs=[_half_spec, _half_spec],
        out_shape=[_half_sds, _half_sds],
    )(h0, deg_out.reshape(NR, 1))


def _norm(alo, ahi, deg_in, deg_out, b):
    return pl.pallas_call(
        _norm_body,
        grid=(NR // _RB,),
        in_specs=[_half_spec, _half_spec, _deg_spec, _deg_spec, _b_spec],
        out_specs=[_half_spec] * 4,
        out_shape=[_half_sds] * 4,
    )(alo, ahi, deg_in.reshape(NR, 1), deg_out.reshape(NR, 1), b.reshape(1, H))


def _norm3(alo, ahi, deg_in, b):
    return pl.pallas_call(
        _norm3_body,
        grid=(NR // _RB,),
        in_specs=[_half_spec, _half_spec, _deg_spec, _b_spec],
        out_specs=[_half_spec, _half_spec, _deg_spec],
        out_shape=[_half_sds, _half_sds,
                   jax.ShapeDtypeStruct((NR, 1), _f32)],
    )(alo, ahi, deg_in.reshape(NR, 1), b.reshape(1, H))


def _topk_body(keys_ref, idx_ref):
    kr, kc = NR // 128, 128
    flat_iota = (lax.broadcasted_iota(jnp.int32, (kr, kc), 0) * kc
                 + lax.broadcasted_iota(jnp.int32, (kr, kc), 1))
    valid = ((flat_iota < HALF)
             | ((flat_iota >= SC_ROWS) & (flat_iota < SC_ROWS + HALF)))
    keys = jnp.where(valid, keys_ref[...], -jnp.inf)
    for k in range(K):
        m = jnp.max(keys)
        idx = jnp.min(jnp.where(keys == m, flat_iota, NR))
        idx_ref[0, k] = idx
        keys = jnp.where(flat_iota == idx, -jnp.inf, keys)
    for k in range(K, CHUNK):
        idx_ref[0, k] = 0


def _topk(keys):
    return pl.pallas_call(
        _topk_body,
        in_specs=[pl.BlockSpec(memory_space=pltpu.VMEM)],
        out_specs=pl.BlockSpec(memory_space=pltpu.SMEM),
        out_shape=jax.ShapeDtypeStruct((1, CHUNK), jnp.int32),
    )(keys)


def _gather_body(idx_hbm, x1l, x1h, x2l, x2h, x3l, x3h, out_hbm,
                 idxv, rows, gsem):
    c = lax.axis_index("c")
    s = lax.axis_index("s")

    @pl.when((c == 0) & (s == 0))
    def _():
        pltpu.sync_copy(idx_hbm.at[0], idxv)
        for q, src in enumerate((x1l, x1h, x2l, x2h, x3l, x3h)):
            pltpu.async_copy(src.at[idxv], rows, gsem).wait()
            pltpu.sync_copy(rows, out_hbm.at[q])


_gather_topk = pl.kernel(
    _gather_body,
    out_type=jax.ShapeDtypeStruct((6, CHUNK, HH), jnp.float32),
    mesh=_mesh,
    compiler_params=pltpu.CompilerParams(use_tc_tiling_on_sc=False),
    scratch_types=[
        pltpu.VMEM((CHUNK,), jnp.int32),
        pltpu.VMEM((CHUNK, HH), jnp.float32),
        pltpu.SemaphoreType.DMA,
    ],
)


def _head_body(tk_ref, w1t_ref, bc1_ref, w2r_ref, bc2_ref, wl1_ref,
               bl1_ref, wl2_ref, bl2_ref, out_ref):
    t = jnp.concatenate([tk_ref[q, 0:K, :] for q in range(6)], axis=1)
    c1 = jnp.maximum(
        jnp.dot(t, w1t_ref[...], preferred_element_type=_f32)
        + bc1_ref[...], 0.0)
    mp = jnp.max(c1.reshape(K // 2, 2, 16), axis=1)
    w = jnp.concatenate([mp[kw:kw + 11, :] for kw in range(5)], axis=1)
    c2 = jnp.maximum(
        jnp.dot(w, w2r_ref[...], preferred_element_type=_f32)
        + bc2_ref[...], 0.0)
    h1 = bl1_ref[...]
    for tpos in range(11):
        h1 = h1 + jnp.dot(c2[tpos:tpos + 1, :], wl1_ref[tpos],
                          preferred_element_type=_f32)
    h1 = jnp.maximum(h1, 0.0)
    out_ref[...] = jnp.sum(h1 * wl2_ref[...]).reshape(1, 1) + bl2_ref[...]


def _head(tk, w1t, bc1, w2r, bc2, wl1p3, bl1, wl2, bl2):
    return pl.pallas_call(
        _head_body,
        out_shape=jax.ShapeDtypeStruct((1, 1), _f32),
    )(tk, w1t, bc1, w2r, bc2, wl1p3, bl1, wl2, bl2)


def kernel(z, edge_index, z_table, b1, b2, b3, w_c1, b_c1, w_c2, b_c2,
           w_l1, b_l1, w_l2, b_l2):
    src = jnp.pad(edge_index[0].astype(jnp.int32), (0, EPAD - E),
                  constant_values=N).reshape(ECHUNKS, CHUNK)
    dst = jnp.pad(edge_index[1].astype(jnp.int32), (0, EPAD - E),
                  constant_values=N).reshape(ECHUNKS, CHUNK)
    zp = jnp.pad(z.astype(jnp.int32), (0, NR - N))

    deg_out, deg_in, srcr, dlc = _prep(src, dst)

    h0 = _embed(zp, z_table)
    hlo, hhi = _norm0(h0, deg_out)

    a1l = _agg(hlo, srcr, dlc)
    a1h = _agg(hhi, srcr, dlc)
    x1l, x1h, hlo, hhi = _norm(a1l, a1h, deg_in, deg_out, b1)
    a2l = _agg(hlo, srcr, dlc)
    a2h = _agg(hhi, srcr, dlc)
    x2l, x2h, hlo, hhi = _norm(a2l, a2h, deg_in, deg_out, b2)
    a3l = _agg(hlo, srcr, dlc)
    a3h = _agg(hhi, srcr, dlc)
    x3l, x3h, keys = _norm3(a3l, a3h, deg_in, b3)

    w1t = w_c1[:, 0, :].T
    w2r = jnp.transpose(w_c2, (2, 1, 0)).reshape(80, 32)
    pos = np.arange(352)
    wl1p3 = w_l1[(pos % 32) * 11 + pos // 32, :].reshape(11, 32, 128)

    idx = _topk(keys.reshape(NR // 128, 128))
    tk = _gather_topk(idx, x1l, x1h, x2l, x2h, x3l, x3h)
    return _head(tk, w1t, b_c1.reshape(1, 16), w2r, b_c2.reshape(1, 32),
                 wl1p3, b_l1.reshape(1, 128), w_l2.reshape(1, 128),
                 b_l2.reshape(1, 1))


# speedup vs baseline: 7.1926x; 2.1041x over previous
import numpy as np

import jax
import jax.numpy as jnp
from jax import lax
from jax.experimental import pallas as pl
from jax.experimental.pallas import tpu as pltpu
from jax.experimental.pallas import tpu_sc as plsc

N = 100000
E = 1600000
H = 32
HH = 16
K = 30
HALF = 50000
SC_ROWS = 51200
NR = 2 * SC_ROWS
TRASH = 50048
ROWS_PT = 3200
CHUNK = 128
CPB = 88
EPAD = 1622016
ECHUNKS = EPAD // CHUNK
NBLK = EPAD // (CPB * CHUNK)
BPT = NBLK // 16
NB = 8

_mesh = plsc.VectorSubcoreMesh(core_axis_name="c", subcore_axis_name="s")


def _map_rows(src_ref, dst_ref, fn):
    def body(r, _):
        for q in range(CHUNK // 16):
            dst_ref[r, pl.ds(q * 16, 16)] = fn(src_ref[r, pl.ds(q * 16, 16)])
        return 0
    lax.fori_loop(0, CPB, body, 0)


def _rho(v):
    r = jnp.where(v >= HALF, v + 1200, v)
    return jnp.where(v >= N, TRASH, r)


def _prep_body(src_hbm, dst_hbm, deg_out_hbm, deg_in_hbm, srcr_hbm, dlc_hbm,
               iblk, ablk, bblk, cblk, ones_v, zero_v, deg_sp,
               ssem, osem, dsem):
    c = lax.axis_index("c")
    s = lax.axis_index("s")

    def zbody(i, _):
        zero_v[pl.ds(i * 16, 16)] = jnp.zeros((16,), jnp.float32)
        return 0
    lax.fori_loop(0, zero_v.shape[0] // 16, zbody, 0, unroll=8)

    def obody(i, _):
        ones_v[pl.ds(i * 16, 16)] = jnp.ones((16,), jnp.float32)
        return 0
    lax.fori_loop(0, CHUNK // 16, obody, 0)

    pltpu.sync_copy(zero_v, deg_sp.at[pl.ds(s * (NR // 16), NR // 16)])
    plsc.subcore_barrier()

    def blk(b, _):
        brow = (s * BPT + b) * CPB

        @pl.when(c == 0)
        def _():
            pltpu.sync_copy(src_hbm.at[pl.ds(brow, CPB)], iblk)
            _map_rows(iblk, ablk, _rho)
            cp = pltpu.async_copy(ablk, srcr_hbm.at[pl.ds(brow, CPB)], osem)
            for j in range(CPB):
                pltpu.async_copy(ones_v, deg_sp.at[ablk.at[j]], dsem,
                                 add=True)
            for j in range(CPB):
                pltpu.make_async_copy(ones_v, deg_sp.at[ablk.at[j]],
                                      dsem).wait()
            cp.wait()

        @pl.when(c != 0)
        def _():
            pltpu.sync_copy(dst_hbm.at[pl.ds(brow, CPB)], iblk)
            _map_rows(iblk, ablk,
                      lambda v: jnp.where(v < HALF, v, HALF + (v & 1023)))
            cp0 = pltpu.async_copy(ablk, dlc_hbm.at[0, pl.ds(brow, CPB)],
                                   osem)
            _map_rows(iblk, bblk,
                      lambda v: jnp.where((v >= HALF) & (v < N), v - HALF,
                                          HALF + (v & 1023)))
            cp1 = pltpu.async_copy(bblk, dlc_hbm.at[1, pl.ds(brow, CPB)],
                                   ssem)
            _map_rows(iblk, cblk, _rho)
            for j in range(CPB):
                pltpu.async_copy(ones_v, deg_sp.at[cblk.at[j]], dsem,
                                 add=True)
            for j in range(CPB):
                pltpu.make_async_copy(ones_v, deg_sp.at[cblk.at[j]],
                                      dsem).wait()
            cp0.wait()
            cp1.wait()
        return 0

    lax.fori_loop(0, BPT, blk, 0)
    plsc.subcore_barrier()

    @pl.when(c == 0)
    def _():
        pltpu.sync_copy(deg_sp.at[pl.ds(s * (NR // 16), NR // 16)],
                        deg_out_hbm.at[pl.ds(s * (NR // 16), NR // 16)])

    @pl.when(c != 0)
    def _():
        pltpu.sync_copy(deg_sp.at[pl.ds(s * (NR // 16), NR // 16)],
                        deg_in_hbm.at[pl.ds(s * (NR // 16), NR // 16)])


_prep = pl.kernel(
    _prep_body,
    out_type=(
        jax.ShapeDtypeStruct((NR,), jnp.float32),
        jax.ShapeDtypeStruct((NR,), jnp.float32),
        jax.ShapeDtypeStruct((ECHUNKS, CHUNK), jnp.int32),
        jax.ShapeDtypeStruct((2, ECHUNKS, CHUNK), jnp.int32),
    ),
    mesh=_mesh,
    compiler_params=pltpu.CompilerParams(use_tc_tiling_on_sc=False),
    scratch_types=[
        pltpu.VMEM((CPB, CHUNK), jnp.int32),
        pltpu.VMEM((CPB, CHUNK), jnp.int32),
        pltpu.VMEM((CPB, CHUNK), jnp.int32),
        pltpu.VMEM((CPB, CHUNK), jnp.int32),
        pltpu.VMEM((CHUNK,), jnp.float32),
        pltpu.VMEM((NR // 16,), jnp.float32),
        pltpu.VMEM_SHARED((NR,), jnp.float32),
        pltpu.SemaphoreType.DMA,
        pltpu.SemaphoreType.DMA,
        pltpu.SemaphoreType.DMA,
    ],
)


def _embed_body(z_hbm, tab_hbm, h0_hbm, zb, rows, gsem):
    c = lax.axis_index("c")
    s = lax.axis_index("s")
    nbase = c * HALF + s * ROWS_PT
    rbase = c * SC_ROWS + s * ROWS_PT

    def body(j, _):
        pltpu.sync_copy(z_hbm.at[pl.ds(nbase + j * CHUNK, CHUNK)], zb)
        pltpu.async_copy(tab_hbm.at[zb], rows, gsem).wait()
        pltpu.sync_copy(rows, h0_hbm.at[pl.ds(rbase + j * CHUNK, CHUNK)])
        return 0

    lax.fori_loop(0, ROWS_PT // CHUNK, body, 0)


_embed = pl.kernel(
    _embed_body,
    out_type=jax.ShapeDtypeStruct((NR, H), jnp.float32),
    mesh=_mesh,
    compiler_params=pltpu.CompilerParams(use_tc_tiling_on_sc=False),
    scratch_types=[
        pltpu.VMEM((CHUNK,), jnp.int32),
        pltpu.VMEM((CHUNK, H), jnp.float32),
        pltpu.SemaphoreType.DMA,
    ],
)


def _agg_body(hn_hbm, srcr_hbm, dlc_hbm, out_hbm,
              sblk, dblk, rows, zb, acc, isem, gsem, ssem):
    c = lax.axis_index("c")
    s = lax.axis_index("s")

    def zbody(i, _):
        zb[i, pl.ds(0, 16)] = jnp.zeros((16,), jnp.float32)
        return 0
    lax.fori_loop(0, 400, zbody, 0, unroll=8)
    for k in range(8):
        pltpu.sync_copy(zb, acc.at[pl.ds(s * ROWS_PT + k * 400, 400)])
    plsc.subcore_barrier()

    cbase = s * BPT * CPB

    def gather(j):
        pltpu.async_copy(hn_hbm.at[sblk.at[j]], rows.at[j % NB], gsem)

    def gwait(j):
        pltpu.make_async_copy(hn_hbm.at[sblk.at[j]], rows.at[j % NB],
                              gsem).wait()

    def scat(j):
        pltpu.async_copy(rows.at[j % NB], acc.at[dblk.at[j]], ssem, add=True)

    def swait(j):
        pltpu.make_async_copy(rows.at[j % NB], acc.at[dblk.at[j]],
                              ssem).wait()

    def blk(b, _):
        crow = cbase + b * CPB
        c1 = pltpu.async_copy(srcr_hbm.at[pl.ds(crow, CPB)], sblk, isem)
        c2 = pltpu.async_copy(dlc_hbm.at[c, pl.ds(crow, CPB)], dblk, isem)
        c1.wait()
        c2.wait()
        for j in range(NB):
            gather(j)
        for j in range(CPB):
            gwait(j)
            scat(j)
            if j + NB < CPB:
                swait(j)
                gather(j + NB)
        for j in range(CPB - NB, CPB):
            swait(j)
        return 0

    lax.fori_loop(0, BPT, blk, 0)

    plsc.subcore_barrier()
    pltpu.sync_copy(acc.at[pl.ds(s * ROWS_PT, ROWS_PT)],
                    out_hbm.at[pl.ds(c * SC_ROWS + s * ROWS_PT, ROWS_PT)])


_agg = pl.kernel(
    _agg_body,
    out_type=jax.ShapeDtypeStruct((NR, HH), jnp.float32),
    mesh=_mesh,
    compiler_params=pltpu.CompilerParams(use_tc_tiling_on_sc=False),
    scratch_types=[
        pltpu.VMEM((CPB, CHUNK), jnp.int32),
        pltpu.VMEM((CPB, CHUNK), jnp.int32),
        pltpu.VMEM((NB, CHUNK, HH), jnp.float32),
        pltpu.VMEM((400, HH), jnp.float32),
        pltpu.VMEM_SHARED((SC_ROWS, HH), jnp.float32),
        pltpu.SemaphoreType.DMA,
        pltpu.SemaphoreType.DMA,
        pltpu.SemaphoreType.DMA,
    ],
)


_RB = 1024


def _norm0_body(h0_ref, dego_ref, lo_ref, hi_ref):
    cout = lax.rsqrt(jnp.maximum(dego_ref[...], 1.0))
    hn = h0_ref[...] * cout
    lo_ref[...] = hn[:, 0:HH]
    hi_ref[...] = hn[:, HH:H]


def _norm_body(alo_ref, ahi_ref, degi_ref, dego_ref, b_ref,
               xlo_ref, xhi_ref, hlo_ref, hhi_ref):
    cin = lax.rsqrt(jnp.maximum(degi_ref[...], 1.0))
    cout = lax.rsqrt(jnp.maximum(dego_ref[...], 1.0))
    xlo = jnp.tanh(alo_ref[...] * cin + b_ref[:, 0:HH])
    xhi = jnp.tanh(ahi_ref[...] * cin + b_ref[:, HH:H])
    xlo_ref[...] = xlo
    xhi_ref[...] = xhi
    hlo_ref[...] = xlo * cout
    hhi_ref[...] = xhi * cout


def _norm3_body(alo_ref, ahi_ref, degi_ref, b_ref, xlo_ref, xhi_ref, key_ref):
    cin = lax.rsqrt(jnp.maximum(degi_ref[...], 1.0))
    xlo_ref[...] = jnp.tanh(alo_ref[...] * cin + b_ref[:, 0:HH])
    xhi = jnp.tanh(ahi_ref[...] * cin + b_ref[:, HH:H])
    xhi_ref[...] = xhi
    key_ref[...] = xhi[:, HH - 1:HH]


_row_spec = pl.BlockSpec((_RB, H), lambda i: (i, 0))
_half_spec = pl.BlockSpec((_RB, HH), lambda i: (i, 0))
_deg_spec = pl.BlockSpec((_RB, 1), lambda i: (i, 0))
_b_spec = pl.BlockSpec((1, H), lambda i: (0, 0))
_f32 = jnp.float32
_half_sds = jax.ShapeDtypeStruct((NR, HH), _f32)


def _norm0(h0, deg_out):
    return pl.pallas_call(
        _norm0_body,
        grid=(NR // _RB,),
        in_specs=[_row_spec, _deg_spec],
        out_specs=[_half_spec, _half_spec],
        out_shape=[_half_sds, _half_sds],
    )(h0, deg_out.reshape(NR, 1))


def _norm(alo, ahi, deg_in, deg_out, b):
    return pl.pallas_call(
        _norm_body,
        grid=(NR // _RB,),
        in_specs=[_half_spec, _half_spec, _deg_spec, _deg_spec, _b_spec],
        out_specs=[_half_spec] * 4,
        out_shape=[_half_sds] * 4,
    )(alo, ahi, deg_in.reshape(NR, 1), deg_out.reshape(NR, 1), b.reshape(1, H))


def _norm3(alo, ahi, deg_in, b):
    return pl.pallas_call(
        _norm3_body,
        grid=(NR // _RB,),
        in_specs=[_half_spec, _half_spec, _deg_spec, _b_spec],
        out_specs=[_half_spec, _half_spec, _deg_spec],
        out_shape=[_half_sds, _half_sds,
                   jax.ShapeDtypeStruct((NR, 1), _f32)],
    )(alo, ahi, deg_in.reshape(NR, 1), b.reshape(1, H))


def _topk_body(keys_ref, idx_ref):
    kr, kc = NR // 128, 128
    flat_iota = (lax.broadcasted_iota(jnp.int32, (kr, kc), 0) * kc
                 + lax.broadcasted_iota(jnp.int32, (kr, kc), 1))
    valid = ((flat_iota < HALF)
             | ((flat_iota >= SC_ROWS) & (flat_iota < SC_ROWS + HALF)))
    keys = jnp.where(valid, keys_ref[...], -jnp.inf)
    for k in range(K):
        m = jnp.max(keys)
        idx = jnp.min(jnp.where(keys == m, flat_iota, NR))
        idx_ref[0, k] = idx
        keys = jnp.where(flat_iota == idx, -jnp.inf, keys)
    for k in range(K, CHUNK):
        idx_ref[0, k] = 0


def _topk(keys):
    return pl.pallas_call(
        _topk_body,
        in_specs=[pl.BlockSpec(memory_space=pltpu.VMEM)],
        out_specs=pl.BlockSpec(memory_space=pltpu.SMEM),
        out_shape=jax.ShapeDtypeStruct((1, CHUNK), jnp.int32),
    )(keys)


def _gather_body(idx_hbm, x1l, x1h, x2l, x2h, x3l, x3h, out_hbm,
                 idxv, rows, gsem):
    c = lax.axis_index("c")
    s = lax.axis_index("s")

    @pl.when((c == 0) & (s == 0))
    def _():
        pltpu.sync_copy(idx_hbm.at[0], idxv)
        for q, src in enumerate((x1l, x1h, x2l, x2h, x3l, x3h)):
            pltpu.async_copy(src.at[idxv], rows, gsem).wait()
            pltpu.sync_copy(rows, out_hbm.at[q])


_gather_topk = pl.kernel(
    _gather_body,
    out_type=jax.ShapeDtypeStruct((6, CHUNK, HH), jnp.float32),
    mesh=_mesh,
    compiler_params=pltpu.CompilerParams(use_tc_tiling_on_sc=False),
    scratch_types=[
        pltpu.VMEM((CHUNK,), jnp.int32),
        pltpu.VMEM((CHUNK, HH), jnp.float32),
        pltpu.SemaphoreType.DMA,
    ],
)


def _head_body(tk_ref, w1t_ref, bc1_ref, w2r_ref, bc2_ref, wl1_ref,
               bl1_ref, wl2_ref, bl2_ref, out_ref):
    t = jnp.concatenate([tk_ref[q, 0:K, :] for q in range(6)], axis=1)
    c1 = jnp.maximum(
        jnp.dot(t, w1t_ref[...], preferred_element_type=_f32)
        + bc1_ref[...], 0.0)
    mp = jnp.max(c1.reshape(K // 2, 2, 16), axis=1)
    w = jnp.concatenate([mp[kw:kw + 11, :] for kw in range(5)], axis=1)
    c2 = jnp.maximum(
        jnp.dot(w, w2r_ref[...], preferred_element_type=_f32)
        + bc2_ref[...], 0.0)
    h1 = bl1_ref[...]
    for tpos in range(11):
        h1 = h1 + jnp.dot(c2[tpos:tpos + 1, :], wl1_ref[tpos],
                          preferred_element_type=_f32)
    h1 = jnp.maximum(h1, 0.0)
    out_ref[...] = jnp.sum(h1 * wl2_ref[...]).reshape(1, 1) + bl2_ref[...]


def _head(tk, w1t, bc1, w2r, bc2, wl1p3, bl1, wl2, bl2):
    return pl.pallas_call(
        _head_body,
        out_shape=jax.ShapeDtypeStruct((1, 1), _f32),
    )(tk, w1t, bc1, w2r, bc2, wl1p3, bl1, wl2, bl2)


def kernel(z, edge_index, z_table, b1, b2, b3, w_c1, b_c1, w_c2, b_c2,
           w_l1, b_l1, w_l2, b_l2):
    src = jnp.pad(edge_index[0].astype(jnp.int32), (0, EPAD - E),
                  constant_values=N).reshape(ECHUNKS, CHUNK)
    dst = jnp.pad(edge_index[1].astype(jnp.int32), (0, EPAD - E),
                  constant_values=N).reshape(ECHUNKS, CHUNK)
    zp = jnp.pad(z.astype(jnp.int32), (0, NR - N))

    deg_out, deg_in, srcr, dlc = _prep(src, dst)

    h0 = _embed(zp, z_table)
    hlo, hhi = _norm0(h0, deg_out)

    a1l = _agg(hlo, srcr, dlc)
    a1h = _agg(hhi, srcr, dlc)
    x1l, x1h, hlo, hhi = _norm(a1l, a1h, deg_in, deg_out, b1)
    a2l = _agg(hlo, srcr, dlc)
    a2h = _agg(hhi, srcr, dlc)
    x2l, x2h, hlo, hhi = _norm(a2l, a2h, deg_in, deg_out, b2)
    a3l = _agg(hlo, srcr, dlc)
    a3h = _agg(hhi, srcr, dlc)
    x3l, x3h, keys = _norm3(a3l, a3h, deg_in, b3)

    w1t = w_c1[:, 0, :].T
    w2r = jnp.transpose(w_c2, (2, 1, 0)).reshape(80, 32)
    pos = np.arange(352)
    wl1p3 = w_l1[(pos % 32) * 11 + pos // 32, :].reshape(11, 32, 128)

    idx = _topk(keys.reshape(NR // 128, 128))
    tk = _gather_topk(idx, x1l, x1h, x2l, x2h, x3l, x3h)
    return _head(tk, w1t, b_c1.reshape(1, 16), w2r, b_c2.reshape(1, 32),
                 wl1p3, b_l1.reshape(1, 128), w_l2.reshape(1, 128),
                 b_l2.reshape(1, 1))
